# baseline (speedup 1.0000x reference)
import jax
import jax.numpy as jnp
from jax import lax
from jax.experimental import pallas as pl
from jax.experimental.pallas import tpu as pltpu
from jax.experimental.pallas import tpu_sc as plsc

_B, _L, _D, _K = 4096, 200, 32, 20
_NC, _NS = 2, 16
_NW = _NC * _NS
_RW = _B // _NW
_NF = 15
_ITEM = 5
_CAT = 6
_NEG = -1e9
_SQD = 5.656854249492381
_VOCABS = (100000, 8, 3, 4, 5, 100000, 1000, 10, 10, 100000, 100000, 100000,
           1000, 5, 24)
_WIDE = tuple(v % 4 == 0 and i != _ITEM for i, v in enumerate(_VOCABS))


def _splat(v):
    return jnp.full((16,), v, jnp.int32)


def _sc_body(*refs):
    idx = refs[0:_NF]
    tab = refs[_NF:2 * _NF]
    seq_ids_hbm = refs[30]
    seq_cat_hbm = refs[31]
    x2d_hbm = refs[32]
    (scat_v, sids_v, cats_v, selt_v, nm_v, gidx_v, retr_v, frow_v, frow4_v,
     tgt_v, int_v, slab_v, fidx_v, fidx4_v, sem, sem2) = refs[33:]

    cid = lax.axis_index("c")
    sid = lax.axis_index("s")
    wid = sid * _NC + cid
    base = wid * _RW

    lane = lax.iota(jnp.int32, 16)

    pltpu.sync_copy(idx[_CAT].at[pl.ds(base, _RW)], cats_v)

    pltpu.async_copy(seq_cat_hbm.at[:, pl.ds(base, 16)], scat_v.at[0], sem2).wait()
    for g in range(_RW // 16):
        b = g % 2
        if g + 1 < _RW // 16:
            cat_nxt = pltpu.async_copy(
                seq_cat_hbm.at[:, pl.ds(base + (g + 1) * 16, 16)],
                scat_v.at[1 - b], sem2)
        ids_cur = pltpu.async_copy(
            seq_ids_hbm.at[:, pl.ds(base + g * 16, 16)], sids_v.at[b], sem2)
        tcat = cats_v[pl.ds(g * 16, 16)]
        for k in range(_K):
            selt_v[pl.ds(k * 16, 16)] = jnp.full((16,), k, jnp.int32)

        def scan_body(p, c):
            pv = _splat(p)
            v = plsc.load_gather(scat_v.at[b], [pv, lane])
            m = (v == tcat) & (c < _K)
            plsc.store_scatter(selt_v, [c * 16 + lane], pv, mask=m)
            return c + jnp.where(m, 1, 0)

        c = lax.fori_loop(0, _L, scan_body, jnp.zeros((16,), jnp.int32))
        nm_v[pl.ds(g * 16, 16)] = c

        ids_cur.wait()
        for k in range(_K):
            pos = selt_v[pl.ds(k * 16, 16)]
            ids = plsc.load_gather(sids_v.at[b], [pos, lane])
            flat = (g * 16 + lane) * _K + k
            plsc.store_scatter(gidx_v, [flat >> 7, flat & 127], ids)
        if g + 1 < _RW // 16:
            cat_nxt.wait()

    descs = [pltpu.async_copy(tab[_ITEM].at[gidx_v.at[q]], retr_v.at[q], sem)
             for q in range(5)]

    def _pack_field(src, r, loff):
        for h in range(2):
            v = plsc.load_gather(src, [_splat(r), 16 * h + lane])
            plsc.store_scatter(slab_v, [_splat(r), _splat(loff + 16 * h) + lane], v)
        return ()

    def _do_field(f):
        loff = (f % 4) * 32
        pltpu.sync_copy(idx[f].at[pl.ds(base, _RW)], fidx_v)
        if _WIDE[f]:
            for i in range(_RW // 16):
                fidx4_v[pl.ds(i * 16, 16)] = fidx_v[pl.ds(i * 16, 16)] >> 2
            pltpu.async_copy(tab[f].at[fidx4_v], frow4_v, sem2).wait()

            def pack_body(r, _):
                rm = (plsc.load_gather(fidx_v, [_splat(r)]) & 3) * _D
                for h in range(2):
                    v = plsc.load_gather(frow4_v, [_splat(r), rm + 16 * h + lane])
                    plsc.store_scatter(
                        slab_v, [_splat(r), _splat(loff + 16 * h) + lane], v)
                return 0
        else:
            dst = tgt_v if f == _ITEM else frow_v
            pltpu.async_copy(tab[f].at[fidx_v], dst, sem2).wait()

            def pack_body(r, _):
                _pack_field(dst, r, loff)
                return 0

        lax.fori_loop(0, _RW, pack_body, 0)

    for cb in range(3):
        for f in range(4 * cb, 4 * cb + 4):
            _do_field(f)
        pltpu.sync_copy(slab_v,
                        x2d_hbm.at[pl.ds(base, _RW), pl.ds(cb * 128, 128)])

    for ch in range(4):
        for d in descs:
            d.wait()

        def attn_body(rl, _):
            r = ch * 32 + rl
            s_lo = jnp.zeros((16,), jnp.float32)
            s_hi = jnp.zeros((16,), jnp.float32)
            flat_lo = _splat(rl * _K) + lane
            flat_hi = jnp.minimum(_splat(rl * _K + 16) + lane, _splat(639))
            for d in range(_D):
                dv = _splat(d)
                c_lo = plsc.load_gather(retr_v, [flat_lo >> 7, flat_lo & 127, dv])
                c_hi = plsc.load_gather(retr_v, [flat_hi >> 7, flat_hi & 127, dv])
                td = plsc.load_gather(tgt_v, [_splat(r), dv])
                s_lo = s_lo + c_lo * td
                s_hi = s_hi + c_hi * td
            nm_r = plsc.load_gather(nm_v, [_splat(r)])
            s_lo = jnp.where(lane < nm_r, s_lo / _SQD, _NEG)
            s_hi = jnp.where(16 + lane < nm_r, s_hi / _SQD,
                             jnp.where(lane < 4, _NEG, -jnp.inf))
            m = jnp.maximum(jnp.max(s_lo), jnp.max(s_hi))
            e_lo = jnp.exp(s_lo - m)
            e_hi = jnp.exp(s_hi - m)
            den = jnp.sum(e_lo) + jnp.sum(e_hi)
            a_lo = e_lo / den
            a_hi = e_hi / den
            i0 = jnp.zeros((16,), jnp.float32)
            i1 = jnp.zeros((16,), jnp.float32)
            for k in range(_K):
                fk = rl * _K + k
                src = a_lo if k < 16 else a_hi
                ak = jnp.take_along_axis(src, _splat(k % 16), axis=0)
                r0 = plsc.load_gather(retr_v, [_splat(fk >> 7), _splat(fk & 127), lane])
                r1 = plsc.load_gather(retr_v, [_splat(fk >> 7), _splat(fk & 127), 16 + lane])
                i0 = i0 + ak * r0
                i1 = i1 + ak * r1
            plsc.store_scatter(int_v, [_splat(r), lane], i0)
            plsc.store_scatter(int_v, [_splat(r), 16 + lane], i1)
            return 0

        lax.fori_loop(0, 32, attn_body, 0)
        if ch + 1 < 4:
            descs = [pltpu.async_copy(
                tab[_ITEM].at[gidx_v.at[(ch + 1) * 5 + q]], retr_v.at[q], sem)
                for q in range(5)]

    for f in (12, 13, 14):
        _do_field(f)

    def pack_int(r, _):
        _pack_field(int_v, r, 96)
        return 0

    lax.fori_loop(0, _RW, pack_int, 0)
    pltpu.sync_copy(slab_v, x2d_hbm.at[pl.ds(base, _RW), pl.ds(384, 128)])


_SC_SCRATCH = [
    pltpu.VMEM((2, _L, 16), jnp.int32),
    pltpu.VMEM((2, _L, 16), jnp.int32),
    pltpu.VMEM((_RW,), jnp.int32),
    pltpu.VMEM((_K * 16,), jnp.int32),
    pltpu.VMEM((_RW,), jnp.int32),
    pltpu.VMEM((_K, 128), jnp.int32),
    pltpu.VMEM((5, 128, _D), jnp.float32),
    pltpu.VMEM((_RW, _D), jnp.float32),
    pltpu.VMEM((_RW, 128), jnp.float32),
    pltpu.VMEM((_RW, _D), jnp.float32),
    pltpu.VMEM((_RW, _D), jnp.float32),
    pltpu.VMEM((_RW, 128), jnp.float32),
    pltpu.VMEM((_RW,), jnp.int32),
    pltpu.VMEM((_RW,), jnp.int32),
    pltpu.SemaphoreType.DMA,
    pltpu.SemaphoreType.DMA,
]

_sc_kernel = pl.kernel(
    _sc_body,
    out_type=jax.ShapeDtypeStruct((_B, 16 * _D), jnp.float32),
    mesh=plsc.VectorSubcoreMesh(core_axis_name="c", subcore_axis_name="s",
                                num_cores=_NC, num_subcores=_NS),
    scratch_types=_SC_SCRATCH,
    compiler_params=pltpu.CompilerParams(use_tc_tiling_on_sc=False,
                                         needs_layout_passes=False),
    name="simcvr_sc",
)

_BB = 512


def _tc_body(x_ref, w1, b1, w2, b2, w3, b3, w4, b4, out_ref):
    x = x_ref[...]
    x = jnp.maximum(jnp.dot(x, w1[...], preferred_element_type=jnp.float32) + b1[...], 0.0)
    x = jnp.maximum(jnp.dot(x, w2[...], preferred_element_type=jnp.float32) + b2[...], 0.0)
    x = jnp.maximum(jnp.dot(x, w3[...], preferred_element_type=jnp.float32) + b3[...], 0.0)
    out_ref[...] = jnp.dot(x, w4[...], preferred_element_type=jnp.float32) + b4[...]


def _full(shape):
    return pl.BlockSpec(shape, lambda i: (0,) * len(shape))


_tc_call = pl.pallas_call(
    _tc_body,
    grid=(_B // _BB,),
    in_specs=([pl.BlockSpec((_BB, 512), lambda i: (i, 0))]
              + [_full((16 * _D, 256)), _full((1, 256)),
                 _full((256, 128)), _full((1, 128)),
                 _full((128, 64)), _full((1, 64)),
                 _full((64, 1)), _full((1, 1))]),
    out_specs=pl.BlockSpec((_BB, 1), lambda i: (i, 0)),
    out_shape=jax.ShapeDtypeStruct((_B, 1), jnp.float32),
    name="simcvr_tc_mlp",
)


@jax.jit
def kernel(user_id, age_level, gender, shopping_level, city_level, item_id,
           item_category, item_price_level, item_sales_level, ad_id,
           campaign_id, customer_id, brand_id, pid, hour,
           seq_item_ids, seq_categories, seq_mask,
           emb_user_id, emb_age_level, emb_gender, emb_shopping_level,
           emb_city_level, emb_item_id, emb_item_category,
           emb_item_price_level, emb_item_sales_level, emb_ad_id,
           emb_campaign_id, emb_customer_id, emb_brand_id, emb_pid,
           emb_hour, W1, b1, W2, b2, W3, b3, W4, b4):
    idxs = (user_id, age_level, gender, shopping_level, city_level, item_id,
            item_category, item_price_level, item_sales_level, ad_id,
            campaign_id, customer_id, brand_id, pid, hour)
    tabs = (emb_user_id, emb_age_level, emb_gender, emb_shopping_level,
            emb_city_level, emb_item_id, emb_item_category,
            emb_item_price_level, emb_item_sales_level, emb_ad_id,
            emb_campaign_id, emb_customer_id, emb_brand_id, emb_pid,
            emb_hour)
    tabs_in = tuple(
        t.reshape(-1, 128) if _WIDE[i] else t for i, t in enumerate(tabs))
    x = _sc_kernel(*idxs, *tabs_in, seq_item_ids.T, seq_categories.T)
    return _tc_call(x, W1, b1.reshape(1, -1), W2, b2.reshape(1, -1),
                    W3, b3.reshape(1, -1), W4, b4.reshape(1, -1))


# speedup vs baseline: 1.0163x; 1.0163x over previous
import jax
import jax.numpy as jnp
from jax import lax
from jax.experimental import pallas as pl
from jax.experimental.pallas import tpu as pltpu
from jax.experimental.pallas import tpu_sc as plsc

_B, _L, _D, _K = 4096, 200, 32, 20
_NC, _NS = 2, 16
_NW = _NC * _NS
_RW = _B // _NW
_NF = 15
_ITEM = 5
_CAT = 6
_NEG = -1e9
_SQD = 5.656854249492381
_VOCABS = (100000, 8, 3, 4, 5, 100000, 1000, 10, 10, 100000, 100000, 100000,
           1000, 5, 24)
_WIDE = tuple(v % 4 == 0 and i != _ITEM for i, v in enumerate(_VOCABS))


def _splat(v):
    return jnp.full((16,), v, jnp.int32)


def _sc_body(*refs):
    idx = refs[0:_NF]
    tab = refs[_NF:2 * _NF]
    seq_ids_hbm = refs[30]
    seq_cat_hbm = refs[31]
    x2d_hbm = refs[32]
    (scat_v, sids_v, cats_v, selt_v, nm_v, gidx_v, retr_v, frow_v, frow4_v,
     tgt_v, int_v, slab_v, fidx_v, fidx4_v, sem, sem2, sem3) = refs[33:]

    cid = lax.axis_index("c")
    sid = lax.axis_index("s")
    wid = sid * _NC + cid
    base = wid * _RW

    lane = lax.iota(jnp.int32, 16)

    pltpu.sync_copy(idx[_CAT].at[pl.ds(base, _RW)], cats_v)

    pltpu.async_copy(seq_cat_hbm.at[pl.ds(base, 16)], scat_v.at[0], sem2).wait()
    for g in range(_RW // 16):
        b = g % 2
        if g + 1 < _RW // 16:
            cat_nxt = pltpu.async_copy(
                seq_cat_hbm.at[pl.ds(base + (g + 1) * 16, 16)],
                scat_v.at[1 - b], sem2)
        ids_cur = pltpu.async_copy(
            seq_ids_hbm.at[pl.ds(base + g * 16, 16)], sids_v.at[b], sem3)
        tcat = cats_v[pl.ds(g * 16, 16)]
        for k in range(_K):
            selt_v[pl.ds(k * 16, 16)] = jnp.full((16,), k, jnp.int32)

        def scan_body(p, c):
            pv = _splat(p)
            v = plsc.load_gather(scat_v.at[b], [lane, pv])
            m = (v == tcat) & (c < _K)
            plsc.store_scatter(selt_v, [c * 16 + lane], pv, mask=m)
            return c + jnp.where(m, 1, 0)

        c = lax.fori_loop(0, _L, scan_body, jnp.zeros((16,), jnp.int32))
        nm_v[pl.ds(g * 16, 16)] = c

        ids_cur.wait()
        for k in range(_K):
            pos = selt_v[pl.ds(k * 16, 16)]
            ids = plsc.load_gather(sids_v.at[b], [lane, pos])
            flat = (g * 16 + lane) * _K + k
            plsc.store_scatter(gidx_v, [flat >> 7, flat & 127], ids)
        if g + 1 < _RW // 16:
            cat_nxt.wait()

    descs = [pltpu.async_copy(tab[_ITEM].at[gidx_v.at[q]], retr_v.at[q], sem)
             for q in range(5)]

    def _pack_field(src, r, loff):
        for h in range(2):
            v = plsc.load_gather(src, [_splat(r), 16 * h + lane])
            plsc.store_scatter(slab_v, [_splat(r), _splat(loff + 16 * h) + lane], v)
        return ()

    def _fire_field(f, s):
        fx = fidx_v.at[s]
        pltpu.sync_copy(idx[f].at[pl.ds(base, _RW)], fx)
        fsem = sem2 if s == 0 else sem3
        if _WIDE[f]:
            fx4 = fidx4_v.at[s]
            for i in range(_RW // 16):
                fx4[pl.ds(i * 16, 16)] = fx[pl.ds(i * 16, 16)] >> 2
            return pltpu.async_copy(tab[f].at[fx4], frow4_v.at[s], fsem)
        dst = tgt_v if f == _ITEM else frow_v.at[s]
        return pltpu.async_copy(tab[f].at[fx], dst, fsem)

    def _pack_one(f, s, desc):
        loff = (f % 4) * 32
        desc.wait()
        if _WIDE[f]:
            def pack_body(r, _):
                rm = (plsc.load_gather(fidx_v.at[s], [_splat(r)]) & 3) * _D
                for h in range(2):
                    v = plsc.load_gather(frow4_v.at[s], [_splat(r), rm + 16 * h + lane])
                    plsc.store_scatter(
                        slab_v, [_splat(r), _splat(loff + 16 * h) + lane], v)
                return 0
        else:
            dst = tgt_v if f == _ITEM else frow_v.at[s]

            def pack_body(r, _):
                _pack_field(dst, r, loff)
                return 0

        lax.fori_loop(0, _RW, pack_body, 0)

    fdesc = _fire_field(0, 0)
    for f in range(12):
        nxt = _fire_field(f + 1, (f + 1) % 2) if f + 1 < 12 else None
        _pack_one(f, f % 2, fdesc)
        fdesc = nxt
        if f % 4 == 3:
            pltpu.sync_copy(
                slab_v, x2d_hbm.at[pl.ds(base, _RW), pl.ds((f // 4) * 128, 128)])

    for ch in range(4):
        for d in descs:
            d.wait()

        def attn_body(rl, _):
            r = ch * 32 + rl
            s_lo = jnp.zeros((16,), jnp.float32)
            s_hi = jnp.zeros((16,), jnp.float32)
            flat_lo = _splat(rl * _K) + lane
            flat_hi = jnp.minimum(_splat(rl * _K + 16) + lane, _splat(639))
            for d in range(_D):
                dv = _splat(d)
                c_lo = plsc.load_gather(retr_v, [flat_lo >> 7, flat_lo & 127, dv])
                c_hi = plsc.load_gather(retr_v, [flat_hi >> 7, flat_hi & 127, dv])
                td = plsc.load_gather(tgt_v, [_splat(r), dv])
                s_lo = s_lo + c_lo * td
                s_hi = s_hi + c_hi * td
            nm_r = plsc.load_gather(nm_v, [_splat(r)])
            s_lo = jnp.where(lane < nm_r, s_lo / _SQD, _NEG)
            s_hi = jnp.where(16 + lane < nm_r, s_hi / _SQD,
                             jnp.where(lane < 4, _NEG, -jnp.inf))
            m = jnp.maximum(jnp.max(s_lo), jnp.max(s_hi))
            e_lo = jnp.exp(s_lo - m)
            e_hi = jnp.exp(s_hi - m)
            den = jnp.sum(e_lo) + jnp.sum(e_hi)
            a_lo = e_lo / den
            a_hi = e_hi / den
            i0 = jnp.zeros((16,), jnp.float32)
            i1 = jnp.zeros((16,), jnp.float32)
            for k in range(_K):
                fk = rl * _K + k
                src = a_lo if k < 16 else a_hi
                ak = jnp.take_along_axis(src, _splat(k % 16), axis=0)
                r0 = plsc.load_gather(retr_v, [_splat(fk >> 7), _splat(fk & 127), lane])
                r1 = plsc.load_gather(retr_v, [_splat(fk >> 7), _splat(fk & 127), 16 + lane])
                i0 = i0 + ak * r0
                i1 = i1 + ak * r1
            plsc.store_scatter(int_v, [_splat(r), lane], i0)
            plsc.store_scatter(int_v, [_splat(r), 16 + lane], i1)
            return 0

        lax.fori_loop(0, 32, attn_body, 0)
        if ch + 1 < 4:
            descs = [pltpu.async_copy(
                tab[_ITEM].at[gidx_v.at[(ch + 1) * 5 + q]], retr_v.at[q], sem)
                for q in range(5)]

    fdesc = _fire_field(12, 0)
    for f in (12, 13, 14):
        nxt = _fire_field(f + 1, (f + 1) % 2) if f < 14 else None
        _pack_one(f, f % 2, fdesc)
        fdesc = nxt

    def pack_int(r, _):
        _pack_field(int_v, r, 96)
        return 0

    lax.fori_loop(0, _RW, pack_int, 0)
    pltpu.sync_copy(slab_v, x2d_hbm.at[pl.ds(base, _RW), pl.ds(384, 128)])


_SC_SCRATCH = [
    pltpu.VMEM((2, 16, _L), jnp.int32),
    pltpu.VMEM((2, 16, _L), jnp.int32),
    pltpu.VMEM((_RW,), jnp.int32),
    pltpu.VMEM((_K * 16,), jnp.int32),
    pltpu.VMEM((_RW,), jnp.int32),
    pltpu.VMEM((_K, 128), jnp.int32),
    pltpu.VMEM((5, 128, _D), jnp.float32),
    pltpu.VMEM((2, _RW, _D), jnp.float32),
    pltpu.VMEM((2, _RW, 128), jnp.float32),
    pltpu.VMEM((_RW, _D), jnp.float32),
    pltpu.VMEM((_RW, _D), jnp.float32),
    pltpu.VMEM((_RW, 128), jnp.float32),
    pltpu.VMEM((2, _RW), jnp.int32),
    pltpu.VMEM((2, _RW), jnp.int32),
    pltpu.SemaphoreType.DMA,
    pltpu.SemaphoreType.DMA,
    pltpu.SemaphoreType.DMA,
]

_sc_kernel = pl.kernel(
    _sc_body,
    out_type=jax.ShapeDtypeStruct((_B, 16 * _D), jnp.float32),
    mesh=plsc.VectorSubcoreMesh(core_axis_name="c", subcore_axis_name="s",
                                num_cores=_NC, num_subcores=_NS),
    scratch_types=_SC_SCRATCH,
    compiler_params=pltpu.CompilerParams(use_tc_tiling_on_sc=False,
                                         needs_layout_passes=False),
    name="simcvr_sc",
)

_BB = 512


def _tc_body(x_ref, w1, b1, w2, b2, w3, b3, w4, b4, out_ref):
    x = x_ref[...]
    x = jnp.maximum(jnp.dot(x, w1[...], preferred_element_type=jnp.float32) + b1[...], 0.0)
    x = jnp.maximum(jnp.dot(x, w2[...], preferred_element_type=jnp.float32) + b2[...], 0.0)
    x = jnp.maximum(jnp.dot(x, w3[...], preferred_element_type=jnp.float32) + b3[...], 0.0)
    out_ref[...] = jnp.dot(x, w4[...], preferred_element_type=jnp.float32) + b4[...]


def _full(shape):
    return pl.BlockSpec(shape, lambda i: (0,) * len(shape))


_tc_call = pl.pallas_call(
    _tc_body,
    grid=(_B // _BB,),
    in_specs=([pl.BlockSpec((_BB, 512), lambda i: (i, 0))]
              + [_full((16 * _D, 256)), _full((1, 256)),
                 _full((256, 128)), _full((1, 128)),
                 _full((128, 64)), _full((1, 64)),
                 _full((64, 1)), _full((1, 1))]),
    out_specs=pl.BlockSpec((_BB, 1), lambda i: (i, 0)),
    out_shape=jax.ShapeDtypeStruct((_B, 1), jnp.float32),
    name="simcvr_tc_mlp",
)


@jax.jit
def kernel(user_id, age_level, gender, shopping_level, city_level, item_id,
           item_category, item_price_level, item_sales_level, ad_id,
           campaign_id, customer_id, brand_id, pid, hour,
           seq_item_ids, seq_categories, seq_mask,
           emb_user_id, emb_age_level, emb_gender, emb_shopping_level,
           emb_city_level, emb_item_id, emb_item_category,
           emb_item_price_level, emb_item_sales_level, emb_ad_id,
           emb_campaign_id, emb_customer_id, emb_brand_id, emb_pid,
           emb_hour, W1, b1, W2, b2, W3, b3, W4, b4):
    idxs = (user_id, age_level, gender, shopping_level, city_level, item_id,
            item_category, item_price_level, item_sales_level, ad_id,
            campaign_id, customer_id, brand_id, pid, hour)
    tabs = (emb_user_id, emb_age_level, emb_gender, emb_shopping_level,
            emb_city_level, emb_item_id, emb_item_category,
            emb_item_price_level, emb_item_sales_level, emb_ad_id,
            emb_campaign_id, emb_customer_id, emb_brand_id, emb_pid,
            emb_hour)
    tabs_in = tuple(
        t.reshape(-1, 128) if _WIDE[i] else t for i, t in enumerate(tabs))
    x = _sc_kernel(*idxs, *tabs_in, seq_item_ids, seq_categories)
    return _tc_call(x, W1, b1.reshape(1, -1), W2, b2.reshape(1, -1),
                    W3, b3.reshape(1, -1), W4, b4.reshape(1, -1))


# speedup vs baseline: 1.3609x; 1.3391x over previous
import jax
import jax.numpy as jnp
from jax import lax
from jax.experimental import pallas as pl
from jax.experimental.pallas import tpu as pltpu
from jax.experimental.pallas import tpu_sc as plsc

_B, _L, _D, _K = 4096, 200, 32, 20
_NC, _NS = 2, 16
_NW = _NC * _NS
_RW = _B // _NW
_NF = 15
_ITEM = 5
_CAT = 6
_NEG = -1e9
_SQD = 5.656854249492381
_VOCABS = (100000, 8, 3, 4, 5, 100000, 1000, 10, 10, 100000, 100000, 100000,
           1000, 5, 24)
_WIDE = (False,) * _NF


def _splat(v):
    return jnp.full((16,), v, jnp.int32)


def _sc_body(*refs):
    idx = refs[0:_NF]
    tab = refs[_NF:2 * _NF]
    seq_ids_hbm = refs[30]
    seq_cat_hbm = refs[31]
    x2d_hbm = refs[32]
    (scat_v, sids_v, cats_v, selt_v, nm_v, gidx_v, retr_v, frow_v, frow4_v,
     tgt_v, int_v, slab_v, fidx_v, fidx4_v, sem, sem2, sem3) = refs[33:]

    cid = lax.axis_index("c")
    sid = lax.axis_index("s")
    wid = sid * _NC + cid
    base = wid * _RW

    lane = lax.iota(jnp.int32, 16)

    pltpu.sync_copy(idx[_CAT].at[pl.ds(base, _RW)], cats_v)

    pltpu.async_copy(seq_cat_hbm.at[pl.ds(base, 16)], scat_v.at[0], sem2).wait()
    for g in range(_RW // 16):
        b = g % 2
        if g + 1 < _RW // 16:
            cat_nxt = pltpu.async_copy(
                seq_cat_hbm.at[pl.ds(base + (g + 1) * 16, 16)],
                scat_v.at[1 - b], sem2)
        ids_cur = pltpu.async_copy(
            seq_ids_hbm.at[pl.ds(base + g * 16, 16)], sids_v.at[b], sem3)
        tcat = cats_v[pl.ds(g * 16, 16)]
        for k in range(_K):
            selt_v[pl.ds(k * 16, 16)] = jnp.full((16,), k, jnp.int32)

        def scan_body(p, c):
            pv = _splat(p)
            v = plsc.load_gather(scat_v.at[b], [lane, pv])
            m = (v == tcat) & (c < _K)
            plsc.store_scatter(selt_v, [c * 16 + lane], pv, mask=m)
            return c + jnp.where(m, 1, 0)

        c = lax.fori_loop(0, _L, scan_body, jnp.zeros((16,), jnp.int32))
        nm_v[pl.ds(g * 16, 16)] = c

        ids_cur.wait()
        for k in range(_K):
            pos = selt_v[pl.ds(k * 16, 16)]
            ids = plsc.load_gather(sids_v.at[b], [lane, pos])
            flat = (g * 16 + lane) * _K + k
            plsc.store_scatter(gidx_v, [flat >> 7, flat & 127], ids)
        if g + 1 < _RW // 16:
            cat_nxt.wait()

    descs = [pltpu.async_copy(tab[_ITEM].at[gidx_v.at[q]], retr_v.at[q], sem)
             for q in range(5)]

    def _pack_field(src, r, loff):
        for h in range(2):
            v = plsc.load_gather(src, [_splat(r), 16 * h + lane])
            plsc.store_scatter(slab_v, [_splat(r), _splat(loff + 16 * h) + lane], v)
        return ()

    def _fire_field(f, s):
        fx = fidx_v.at[s]
        pltpu.sync_copy(idx[f].at[pl.ds(base, _RW)], fx)
        fsem = sem2 if s == 0 else sem3
        if _WIDE[f]:
            fx4 = fidx4_v.at[s]
            for i in range(_RW // 16):
                fx4[pl.ds(i * 16, 16)] = fx[pl.ds(i * 16, 16)] >> 2
            return pltpu.async_copy(tab[f].at[fx4], frow4_v.at[s], fsem)
        dst = tgt_v if f == _ITEM else frow_v.at[s]
        return pltpu.async_copy(tab[f].at[fx], dst, fsem)

    def _pack_one(f, s, desc):
        loff = (f % 4) * 32
        desc.wait()
        if _WIDE[f]:
            def pack_body(r, _):
                rm = (plsc.load_gather(fidx_v.at[s], [_splat(r)]) & 3) * _D
                for h in range(2):
                    v = plsc.load_gather(frow4_v.at[s], [_splat(r), rm + 16 * h + lane])
                    plsc.store_scatter(
                        slab_v, [_splat(r), _splat(loff + 16 * h) + lane], v)
                return 0
        else:
            dst = tgt_v if f == _ITEM else frow_v.at[s]

            def pack_body(r, _):
                _pack_field(dst, r, loff)
                return 0

        lax.fori_loop(0, _RW, pack_body, 0)

    fdesc = _fire_field(0, 0)
    for f in range(12):
        nxt = _fire_field(f + 1, (f + 1) % 2) if f + 1 < 12 else None
        _pack_one(f, f % 2, fdesc)
        fdesc = nxt
        if f % 4 == 3:
            pltpu.sync_copy(
                slab_v, x2d_hbm.at[pl.ds(base, _RW), pl.ds((f // 4) * 128, 128)])

    for ch in range(4):
        for d in descs:
            d.wait()

        def attn_body(rl, _):
            r = ch * 32 + rl
            s_lo = jnp.zeros((16,), jnp.float32)
            s_hi = jnp.zeros((16,), jnp.float32)
            flat_lo = _splat(rl * _K) + lane
            flat_hi = jnp.minimum(_splat(rl * _K + 16) + lane, _splat(639))
            for d in range(_D):
                dv = _splat(d)
                c_lo = plsc.load_gather(retr_v, [flat_lo >> 7, flat_lo & 127, dv])
                c_hi = plsc.load_gather(retr_v, [flat_hi >> 7, flat_hi & 127, dv])
                td = plsc.load_gather(tgt_v, [_splat(r), dv])
                s_lo = s_lo + c_lo * td
                s_hi = s_hi + c_hi * td
            nm_r = plsc.load_gather(nm_v, [_splat(r)])
            s_lo = jnp.where(lane < nm_r, s_lo / _SQD, _NEG)
            s_hi = jnp.where(16 + lane < nm_r, s_hi / _SQD,
                             jnp.where(lane < 4, _NEG, -jnp.inf))
            m = jnp.maximum(jnp.max(s_lo), jnp.max(s_hi))
            e_lo = jnp.exp(s_lo - m)
            e_hi = jnp.exp(s_hi - m)
            den = jnp.sum(e_lo) + jnp.sum(e_hi)
            a_lo = e_lo / den
            a_hi = e_hi / den
            i0 = jnp.zeros((16,), jnp.float32)
            i1 = jnp.zeros((16,), jnp.float32)
            for k in range(_K):
                fk = rl * _K + k
                src = a_lo if k < 16 else a_hi
                ak = jnp.take_along_axis(src, _splat(k % 16), axis=0)
                r0 = plsc.load_gather(retr_v, [_splat(fk >> 7), _splat(fk & 127), lane])
                r1 = plsc.load_gather(retr_v, [_splat(fk >> 7), _splat(fk & 127), 16 + lane])
                i0 = i0 + ak * r0
                i1 = i1 + ak * r1
            plsc.store_scatter(int_v, [_splat(r), lane], i0)
            plsc.store_scatter(int_v, [_splat(r), 16 + lane], i1)
            return 0

        lax.fori_loop(0, 32, attn_body, 0)
        if ch + 1 < 4:
            descs = [pltpu.async_copy(
                tab[_ITEM].at[gidx_v.at[(ch + 1) * 5 + q]], retr_v.at[q], sem)
                for q in range(5)]

    fdesc = _fire_field(12, 0)
    for f in (12, 13, 14):
        nxt = _fire_field(f + 1, (f + 1) % 2) if f < 14 else None
        _pack_one(f, f % 2, fdesc)
        fdesc = nxt

    def pack_int(r, _):
        _pack_field(int_v, r, 96)
        return 0

    lax.fori_loop(0, _RW, pack_int, 0)
    pltpu.sync_copy(slab_v, x2d_hbm.at[pl.ds(base, _RW), pl.ds(384, 128)])


_SC_SCRATCH = [
    pltpu.VMEM((2, 16, _L), jnp.int32),
    pltpu.VMEM((2, 16, _L), jnp.int32),
    pltpu.VMEM((_RW,), jnp.int32),
    pltpu.VMEM((_K * 16,), jnp.int32),
    pltpu.VMEM((_RW,), jnp.int32),
    pltpu.VMEM((_K, 128), jnp.int32),
    pltpu.VMEM((5, 128, _D), jnp.float32),
    pltpu.VMEM((2, _RW, _D), jnp.float32),
    pltpu.VMEM((2, _RW, 128), jnp.float32),
    pltpu.VMEM((_RW, _D), jnp.float32),
    pltpu.VMEM((_RW, _D), jnp.float32),
    pltpu.VMEM((_RW, 128), jnp.float32),
    pltpu.VMEM((2, _RW), jnp.int32),
    pltpu.VMEM((2, _RW), jnp.int32),
    pltpu.SemaphoreType.DMA,
    pltpu.SemaphoreType.DMA,
    pltpu.SemaphoreType.DMA,
]

_sc_kernel = pl.kernel(
    _sc_body,
    out_type=jax.ShapeDtypeStruct((_B, 16 * _D), jnp.float32),
    mesh=plsc.VectorSubcoreMesh(core_axis_name="c", subcore_axis_name="s",
                                num_cores=_NC, num_subcores=_NS),
    scratch_types=_SC_SCRATCH,
    compiler_params=pltpu.CompilerParams(use_tc_tiling_on_sc=False,
                                         needs_layout_passes=False),
    name="simcvr_sc",
)

_BB = 512


def _tc_body(x_ref, w1, b1, w2, b2, w3, b3, w4, b4, out_ref):
    x = x_ref[...]
    x = jnp.maximum(jnp.dot(x, w1[...], preferred_element_type=jnp.float32) + b1[...], 0.0)
    x = jnp.maximum(jnp.dot(x, w2[...], preferred_element_type=jnp.float32) + b2[...], 0.0)
    x = jnp.maximum(jnp.dot(x, w3[...], preferred_element_type=jnp.float32) + b3[...], 0.0)
    out_ref[...] = jnp.dot(x, w4[...], preferred_element_type=jnp.float32) + b4[...]


def _full(shape):
    return pl.BlockSpec(shape, lambda i: (0,) * len(shape))


_tc_call = pl.pallas_call(
    _tc_body,
    grid=(_B // _BB,),
    in_specs=([pl.BlockSpec((_BB, 512), lambda i: (i, 0))]
              + [_full((16 * _D, 256)), _full((1, 256)),
                 _full((256, 128)), _full((1, 128)),
                 _full((128, 64)), _full((1, 64)),
                 _full((64, 1)), _full((1, 1))]),
    out_specs=pl.BlockSpec((_BB, 1), lambda i: (i, 0)),
    out_shape=jax.ShapeDtypeStruct((_B, 1), jnp.float32),
    name="simcvr_tc_mlp",
)


@jax.jit
def kernel(user_id, age_level, gender, shopping_level, city_level, item_id,
           item_category, item_price_level, item_sales_level, ad_id,
           campaign_id, customer_id, brand_id, pid, hour,
           seq_item_ids, seq_categories, seq_mask,
           emb_user_id, emb_age_level, emb_gender, emb_shopping_level,
           emb_city_level, emb_item_id, emb_item_category,
           emb_item_price_level, emb_item_sales_level, emb_ad_id,
           emb_campaign_id, emb_customer_id, emb_brand_id, emb_pid,
           emb_hour, W1, b1, W2, b2, W3, b3, W4, b4):
    idxs = (user_id, age_level, gender, shopping_level, city_level, item_id,
            item_category, item_price_level, item_sales_level, ad_id,
            campaign_id, customer_id, brand_id, pid, hour)
    tabs = (emb_user_id, emb_age_level, emb_gender, emb_shopping_level,
            emb_city_level, emb_item_id, emb_item_category,
            emb_item_price_level, emb_item_sales_level, emb_ad_id,
            emb_campaign_id, emb_customer_id, emb_brand_id, emb_pid,
            emb_hour)
    tabs_in = tuple(
        t.reshape(-1, 128) if _WIDE[i] else t for i, t in enumerate(tabs))
    x = _sc_kernel(*idxs, *tabs_in, seq_item_ids, seq_categories)
    return _tc_call(x, W1, b1.reshape(1, -1), W2, b2.reshape(1, -1),
                    W3, b3.reshape(1, -1), W4, b4.reshape(1, -1))


# speedup vs baseline: 1.3627x; 1.0013x over previous
import jax
import jax.numpy as jnp
from jax import lax
from jax.experimental import pallas as pl
from jax.experimental.pallas import tpu as pltpu
from jax.experimental.pallas import tpu_sc as plsc

_B, _L, _D, _K = 4096, 200, 32, 20
_NC, _NS = 2, 16
_NW = _NC * _NS
_RW = _B // _NW
_NF = 15
_ITEM = 5
_CAT = 6
_NEG = -1e9
_SQD = 5.656854249492381
_VOCABS = (100000, 8, 3, 4, 5, 100000, 1000, 10, 10, 100000, 100000, 100000,
           1000, 5, 24)
_WIDE = (False,) * _NF


def _splat(v):
    return jnp.full((16,), v, jnp.int32)


def _sc_body(*refs):
    idx = refs[0:_NF]
    tab = refs[_NF:2 * _NF]
    seq_ids_hbm = refs[30]
    seq_cat_hbm = refs[31]
    x2d_hbm = refs[32]
    (scat_v, sids_v, cats_v, selt_v, nm_v, gidx_v, retr_v, frow_v, frow4_v,
     tgt_v, int_v, slab_v, fidx_v, fidx4_v, sem, sem2, sem3) = refs[33:]

    cid = lax.axis_index("c")
    sid = lax.axis_index("s")
    wid = sid * _NC + cid
    base = wid * _RW

    lane = lax.iota(jnp.int32, 16)

    pltpu.sync_copy(idx[_CAT].at[pl.ds(base, _RW)], cats_v)

    pltpu.async_copy(seq_cat_hbm.at[pl.ds(base, 16)], scat_v.at[0], sem2).wait()
    for g in range(_RW // 16):
        b = g % 2
        if g + 1 < _RW // 16:
            cat_nxt = pltpu.async_copy(
                seq_cat_hbm.at[pl.ds(base + (g + 1) * 16, 16)],
                scat_v.at[1 - b], sem2)
        ids_cur = pltpu.async_copy(
            seq_ids_hbm.at[pl.ds(base + g * 16, 16)], sids_v.at[b], sem3)
        tcat = cats_v[pl.ds(g * 16, 16)]
        for k in range(_K):
            selt_v[pl.ds(k * 16, 16)] = jnp.full((16,), k, jnp.int32)

        def scan_body(p, c):
            pv = _splat(p)
            v = plsc.load_gather(scat_v.at[b], [lane, pv])
            m = (v == tcat) & (c < _K)
            plsc.store_scatter(selt_v, [c * 16 + lane], pv, mask=m)
            return c + jnp.where(m, 1, 0)

        c = lax.fori_loop(0, _L, scan_body, jnp.zeros((16,), jnp.int32))
        nm_v[pl.ds(g * 16, 16)] = c

        ids_cur.wait()
        for k in range(_K):
            pos = selt_v[pl.ds(k * 16, 16)]
            ids = plsc.load_gather(sids_v.at[b], [lane, pos])
            flat = (g * 16 + lane) * _K + k
            plsc.store_scatter(gidx_v, [flat >> 7, flat & 127], ids)
        if g + 1 < _RW // 16:
            cat_nxt.wait()

    descs = [pltpu.async_copy(tab[_ITEM].at[gidx_v.at[q]], retr_v.at[q], sem)
             for q in range(5)]

    def _pack_field(src, r, loff):
        for h in range(2):
            v = plsc.load_gather(src, [_splat(r), 16 * h + lane])
            plsc.store_scatter(slab_v, [_splat(r), _splat(loff + 16 * h) + lane], v)
        return ()

    def _fire_field(f, s):
        fx = fidx_v.at[s]
        pltpu.sync_copy(idx[f].at[pl.ds(base, _RW)], fx)
        fsem = sem2 if s == 0 else sem3
        if _WIDE[f]:
            fx4 = fidx4_v.at[s]
            for i in range(_RW // 16):
                fx4[pl.ds(i * 16, 16)] = fx[pl.ds(i * 16, 16)] >> 2
            return pltpu.async_copy(tab[f].at[fx4], frow4_v.at[s], fsem)
        dst = tgt_v if f == _ITEM else frow_v.at[s]
        return pltpu.async_copy(tab[f].at[fx], dst, fsem)

    def _pack_one(f, s, desc):
        loff = (f % 4) * 32
        desc.wait()
        if _WIDE[f]:
            def pack_body(r, _):
                rm = (plsc.load_gather(fidx_v.at[s], [_splat(r)]) & 3) * _D
                for h in range(2):
                    v = plsc.load_gather(frow4_v.at[s], [_splat(r), rm + 16 * h + lane])
                    plsc.store_scatter(
                        slab_v, [_splat(r), _splat(loff + 16 * h) + lane], v)
                return 0
        else:
            dst = tgt_v if f == _ITEM else frow_v.at[s]

            def pack_body(r, _):
                _pack_field(dst, r, loff)
                return 0

        lax.fori_loop(0, _RW, pack_body, 0)

    fdesc = _fire_field(0, 0)
    for f in range(12):
        nxt = _fire_field(f + 1, (f + 1) % 2) if f + 1 < 12 else None
        _pack_one(f, f % 2, fdesc)
        fdesc = nxt
        if f % 4 == 3:
            pltpu.sync_copy(
                slab_v, x2d_hbm.at[pl.ds(base, _RW), pl.ds((f // 4) * 128, 128)])

    for ch in range(4):
        for d in descs:
            d.wait()

        def attn_body(rl, _):
            r = ch * 32 + rl
            s_lo = jnp.zeros((16,), jnp.float32)
            s_hi = jnp.zeros((16,), jnp.float32)
            flat_lo = _splat(rl * _K) + lane
            flat_hi = jnp.minimum(_splat(rl * _K + 16) + lane, _splat(639))
            t0 = plsc.load_gather(tgt_v, [_splat(r), lane])
            t1 = plsc.load_gather(tgt_v, [_splat(r), 16 + lane])
            for d in range(_D):
                dv = _splat(d)
                c_lo = plsc.load_gather(retr_v, [flat_lo >> 7, flat_lo & 127, dv])
                c_hi = plsc.load_gather(retr_v, [flat_hi >> 7, flat_hi & 127, dv])
                td = jnp.take_along_axis(t0 if d < 16 else t1, _splat(d % 16), axis=0)
                s_lo = s_lo + c_lo * td
                s_hi = s_hi + c_hi * td
            nm_r = plsc.load_gather(nm_v, [_splat(r)])
            s_lo = jnp.where(lane < nm_r, s_lo / _SQD, _NEG)
            s_hi = jnp.where(16 + lane < nm_r, s_hi / _SQD,
                             jnp.where(lane < 4, _NEG, -jnp.inf))
            m = jnp.maximum(jnp.max(s_lo), jnp.max(s_hi))
            e_lo = jnp.exp(s_lo - m)
            e_hi = jnp.exp(s_hi - m)
            den = jnp.sum(e_lo) + jnp.sum(e_hi)
            a_lo = e_lo / den
            a_hi = e_hi / den
            i0 = jnp.zeros((16,), jnp.float32)
            i1 = jnp.zeros((16,), jnp.float32)
            for k in range(_K):
                fk = rl * _K + k
                src = a_lo if k < 16 else a_hi
                ak = jnp.take_along_axis(src, _splat(k % 16), axis=0)
                r0 = plsc.load_gather(retr_v, [_splat(fk >> 7), _splat(fk & 127), lane])
                r1 = plsc.load_gather(retr_v, [_splat(fk >> 7), _splat(fk & 127), 16 + lane])
                i0 = i0 + ak * r0
                i1 = i1 + ak * r1
            plsc.store_scatter(int_v, [_splat(r), lane], i0)
            plsc.store_scatter(int_v, [_splat(r), 16 + lane], i1)
            return 0

        lax.fori_loop(0, 32, attn_body, 0)
        if ch + 1 < 4:
            descs = [pltpu.async_copy(
                tab[_ITEM].at[gidx_v.at[(ch + 1) * 5 + q]], retr_v.at[q], sem)
                for q in range(5)]

    fdesc = _fire_field(12, 0)
    for f in (12, 13, 14):
        nxt = _fire_field(f + 1, (f + 1) % 2) if f < 14 else None
        _pack_one(f, f % 2, fdesc)
        fdesc = nxt

    def pack_int(r, _):
        _pack_field(int_v, r, 96)
        return 0

    lax.fori_loop(0, _RW, pack_int, 0)
    pltpu.sync_copy(slab_v, x2d_hbm.at[pl.ds(base, _RW), pl.ds(384, 128)])


_SC_SCRATCH = [
    pltpu.VMEM((2, 16, _L), jnp.int32),
    pltpu.VMEM((2, 16, _L), jnp.int32),
    pltpu.VMEM((_RW,), jnp.int32),
    pltpu.VMEM((_K * 16,), jnp.int32),
    pltpu.VMEM((_RW,), jnp.int32),
    pltpu.VMEM((_K, 128), jnp.int32),
    pltpu.VMEM((5, 128, _D), jnp.float32),
    pltpu.VMEM((2, _RW, _D), jnp.float32),
    pltpu.VMEM((2, _RW, 128), jnp.float32),
    pltpu.VMEM((_RW, _D), jnp.float32),
    pltpu.VMEM((_RW, _D), jnp.float32),
    pltpu.VMEM((_RW, 128), jnp.float32),
    pltpu.VMEM((2, _RW), jnp.int32),
    pltpu.VMEM((2, _RW), jnp.int32),
    pltpu.SemaphoreType.DMA,
    pltpu.SemaphoreType.DMA,
    pltpu.SemaphoreType.DMA,
]

_sc_kernel = pl.kernel(
    _sc_body,
    out_type=jax.ShapeDtypeStruct((_B, 16 * _D), jnp.float32),
    mesh=plsc.VectorSubcoreMesh(core_axis_name="c", subcore_axis_name="s",
                                num_cores=_NC, num_subcores=_NS),
    scratch_types=_SC_SCRATCH,
    compiler_params=pltpu.CompilerParams(use_tc_tiling_on_sc=False,
                                         needs_layout_passes=False),
    name="simcvr_sc",
)

_BB = 512


def _tc_body(x_ref, w1, b1, w2, b2, w3, b3, w4, b4, out_ref):
    x = x_ref[...]
    x = jnp.maximum(jnp.dot(x, w1[...], preferred_element_type=jnp.float32) + b1[...], 0.0)
    x = jnp.maximum(jnp.dot(x, w2[...], preferred_element_type=jnp.float32) + b2[...], 0.0)
    x = jnp.maximum(jnp.dot(x, w3[...], preferred_element_type=jnp.float32) + b3[...], 0.0)
    out_ref[...] = jnp.dot(x, w4[...], preferred_element_type=jnp.float32) + b4[...]


def _full(shape):
    return pl.BlockSpec(shape, lambda i: (0,) * len(shape))


_tc_call = pl.pallas_call(
    _tc_body,
    grid=(_B // _BB,),
    in_specs=([pl.BlockSpec((_BB, 512), lambda i: (i, 0))]
              + [_full((16 * _D, 256)), _full((1, 256)),
                 _full((256, 128)), _full((1, 128)),
                 _full((128, 64)), _full((1, 64)),
                 _full((64, 1)), _full((1, 1))]),
    out_specs=pl.BlockSpec((_BB, 1), lambda i: (i, 0)),
    out_shape=jax.ShapeDtypeStruct((_B, 1), jnp.float32),
    name="simcvr_tc_mlp",
)


@jax.jit
def kernel(user_id, age_level, gender, shopping_level, city_level, item_id,
           item_category, item_price_level, item_sales_level, ad_id,
           campaign_id, customer_id, brand_id, pid, hour,
           seq_item_ids, seq_categories, seq_mask,
           emb_user_id, emb_age_level, emb_gender, emb_shopping_level,
           emb_city_level, emb_item_id, emb_item_category,
           emb_item_price_level, emb_item_sales_level, emb_ad_id,
           emb_campaign_id, emb_customer_id, emb_brand_id, emb_pid,
           emb_hour, W1, b1, W2, b2, W3, b3, W4, b4):
    idxs = (user_id, age_level, gender, shopping_level, city_level, item_id,
            item_category, item_price_level, item_sales_level, ad_id,
            campaign_id, customer_id, brand_id, pid, hour)
    tabs = (emb_user_id, emb_age_level, emb_gender, emb_shopping_level,
            emb_city_level, emb_item_id, emb_item_category,
            emb_item_price_level, emb_item_sales_level, emb_ad_id,
            emb_campaign_id, emb_customer_id, emb_brand_id, emb_pid,
            emb_hour)
    tabs_in = tuple(
        t.reshape(-1, 128) if _WIDE[i] else t for i, t in enumerate(tabs))
    x = _sc_kernel(*idxs, *tabs_in, seq_item_ids, seq_categories)
    return _tc_call(x, W1, b1.reshape(1, -1), W2, b2.reshape(1, -1),
                    W3, b3.reshape(1, -1), W4, b4.reshape(1, -1))
